# baseline (speedup 1.0000x reference)
import functools

import jax
import jax.numpy as jnp
from jax.experimental import pallas as pl
from jax.experimental.pallas import tpu as pltpu


def _bq_matmul_kernel(x_hbm, binary_ref, scale_ref, bias_ref, out_ref, x_v, sem):
    j = pl.program_id(0)

    @pl.when(j == 0)
    def _start_x():
        pltpu.make_async_copy(x_hbm, x_v, sem).start()

    nbits, g, blk = binary_ref.shape
    shifts = 7 - jax.lax.broadcasted_iota(jnp.int32, (1, 8, 1), 1)
    acc = None
    for b in range(nbits):
        byte = binary_ref[b]
        bits = (byte[:, None, :] >> shifts) & 1
        fb = bits.astype(jnp.float32) * scale_ref[b][:, None, :]
        acc = fb if acc is None else acc + fb
    csum = jnp.sum(scale_ref[:, 0, :], axis=0)
    w = 2.0 * acc - csum[None, None, :]
    weff = w.reshape(g * 8, blk).astype(jnp.bfloat16)

    @pl.when(j == 0)
    def _wait_x():
        pltpu.make_async_copy(x_hbm, x_v, sem).wait()

    xb = x_v[...].astype(jnp.bfloat16)
    out = jnp.dot(xb, weff, preferred_element_type=jnp.float32)
    out_ref[...] = out + bias_ref[...]


@functools.partial(jax.jit, static_argnames=())
def kernel(x, binary, scale, bias):
    size_out = x.shape[:-1] + (bias.shape[-1],)
    x2 = x.reshape(-1, x.shape[-1])
    t, nx = x2.shape
    nbits = scale.shape[1]
    nf = scale.shape[2]
    g = nx // 8
    binary3 = binary.reshape(nbits, g, nf)
    scale3 = scale.reshape(nbits, 1, nf)
    bias2 = bias.reshape(1, nf)
    blk = 256 if nf % 256 == 0 else nf
    out = pl.pallas_call(
        _bq_matmul_kernel,
        grid=(nf // blk,),
        in_specs=[
            pl.BlockSpec(memory_space=pltpu.MemorySpace.HBM),
            pl.BlockSpec((nbits, g, blk), lambda j: (0, 0, j)),
            pl.BlockSpec((nbits, 1, blk), lambda j: (0, 0, j)),
            pl.BlockSpec((1, blk), lambda j: (0, j)),
        ],
        out_specs=pl.BlockSpec((t, blk), lambda j: (0, j)),
        out_shape=jax.ShapeDtypeStruct((t, nf), jnp.float32),
        scratch_shapes=[
            pltpu.VMEM((t, nx), jnp.float32),
            pltpu.SemaphoreType.DMA,
        ],
    )(x2, binary3, scale3, bias2)
    return out.reshape(size_out)


# speedup vs baseline: 1.6645x; 1.6645x over previous
import functools

import jax
import jax.numpy as jnp
from jax.experimental import pallas as pl


def _bq_matmul_kernel(x_ref, binary_ref, scale_ref, bias_ref, out_ref):
    nbits, g, nf = binary_ref.shape
    shifts = 7 - jax.lax.broadcasted_iota(jnp.int32, (1, 8, 1), 1)
    acc = None
    for b in range(nbits):
        byte = binary_ref[b]
        bits = (byte[:, None, :] >> shifts) & 1
        fb = bits.astype(jnp.float32) * scale_ref[b][None, None, :]
        acc = fb if acc is None else acc + fb
    csum = jnp.sum(scale_ref[...], axis=0)
    w = 2.0 * acc - csum[None, None, :]
    weff = w.reshape(g * 8, nf).astype(jnp.bfloat16)
    xb = x_ref[...].astype(jnp.bfloat16)
    out = jnp.dot(xb, weff, preferred_element_type=jnp.float32)
    out_ref[...] = out + bias_ref[...]


@functools.partial(jax.jit, static_argnames=())
def kernel(x, binary, scale, bias):
    size_out = x.shape[:-1] + (bias.shape[-1],)
    x2 = x.reshape(-1, x.shape[-1])
    t, nx = x2.shape
    nbits = scale.shape[1]
    nf = scale.shape[2]
    g = nx // 8
    binary3 = binary.reshape(nbits, g, nf)
    scale2 = scale.reshape(nbits, nf)
    bias2 = bias.reshape(1, nf)
    out = pl.pallas_call(
        _bq_matmul_kernel,
        out_shape=jax.ShapeDtypeStruct((t, nf), jnp.float32),
    )(x2, binary3, scale2, bias2)
    return out.reshape(size_out)


# speedup vs baseline: 1.9600x; 1.1775x over previous
import functools

import jax
import jax.numpy as jnp
from jax.experimental import pallas as pl


def _bq_matmul_kernel(x_ref, binary_ref, scale_ref, bias_ref, out_ref):
    nbits, g, nf = binary_ref.shape
    combo = binary_ref[0]
    for b in range(1, nbits):
        combo = combo | (binary_ref[b] << (8 * b))
    ncombo = (~combo)[:, None, :]
    ii = jax.lax.broadcasted_iota(jnp.int32, (1, 8, 1), 1)
    signbit = jnp.int32(-2**31)
    w = None
    for b in range(nbits):
        flip = (ncombo << (24 - 8 * b + ii)) & signbit
        sint = jax.lax.bitcast_convert_type(scale_ref[b], jnp.int32)
        wb = jax.lax.bitcast_convert_type(flip ^ sint[None, None, :],
                                          jnp.float32)
        w = wb if w is None else w + wb
    weff = w.reshape(g * 8, nf).astype(jnp.bfloat16)
    xb = x_ref[...].astype(jnp.bfloat16)
    out = jnp.dot(xb, weff, preferred_element_type=jnp.float32)
    out_ref[...] = out + bias_ref[...]


@functools.partial(jax.jit, static_argnames=())
def kernel(x, binary, scale, bias):
    size_out = x.shape[:-1] + (bias.shape[-1],)
    x2 = x.reshape(-1, x.shape[-1])
    t, nx = x2.shape
    nbits = scale.shape[1]
    nf = scale.shape[2]
    g = nx // 8
    binary3 = binary.reshape(nbits, g, nf)
    scale2 = scale.reshape(nbits, nf)
    bias2 = bias.reshape(1, nf)
    out = pl.pallas_call(
        _bq_matmul_kernel,
        out_shape=jax.ShapeDtypeStruct((t, nf), jnp.float32),
    )(x2, binary3, scale2, bias2)
    return out.reshape(size_out)
